# baseline (speedup 1.0000x reference)
import jax
import jax.numpy as jnp
from jax.experimental import pallas as pl
from jax.experimental.pallas import tpu as pltpu

_IPB = 4


def _cc_body(cam_ref, w_ref, b_ref, tex_ref, out_ref):
    i = pl.program_id(0)
    j = pl.program_id(1)
    ci = cam_ref[i * _IPB + j]
    for c in range(3):
        w = w_ref[ci, c]
        b = b_ref[ci, c]
        out_ref[0, c] = tex_ref[j, c] * w + b


@jax.jit
def kernel(texture, cam, weight, bias):
    B, C, H, W = texture.shape
    dt = texture.dtype
    w_full = jnp.concatenate(
        [jnp.ones((1, C), dt), weight.reshape(-1, C)], axis=0)
    b_full = jnp.concatenate(
        [jnp.zeros((1, C), dt), bias.reshape(-1, C)], axis=0)
    cam32 = cam.astype(jnp.int32)
    return pl.pallas_call(
        _cc_body,
        grid=(B // _IPB, _IPB),
        in_specs=[
            pl.BlockSpec(memory_space=pltpu.SMEM),
            pl.BlockSpec(memory_space=pltpu.SMEM),
            pl.BlockSpec(memory_space=pltpu.SMEM),
            pl.BlockSpec((_IPB, C, H, W), lambda i, j: (i, 0, 0, 0)),
        ],
        out_specs=pl.BlockSpec((1, C, H, W), lambda i, j: (i * _IPB + j, 0, 0, 0)),
        out_shape=jax.ShapeDtypeStruct(texture.shape, dt),
        compiler_params=pltpu.CompilerParams(
            dimension_semantics=("arbitrary", "arbitrary")),
    )(cam32, w_full, b_full, texture)


# speedup vs baseline: 1.1891x; 1.1891x over previous
import jax
import jax.numpy as jnp
from jax.experimental import pallas as pl
from jax.experimental.pallas import tpu as pltpu

_IPB = 4


def _cc_body(cam_ref, w_ref, b_ref, tex_ref, out_ref):
    i = pl.program_id(0)
    for k in range(_IPB):
        ci = cam_ref[i * _IPB + k]
        for c in range(3):
            w = w_ref[ci, c]
            b = b_ref[ci, c]
            out_ref[k, c] = tex_ref[k, c] * w + b


@jax.jit
def kernel(texture, cam, weight, bias):
    B, C, H, W = texture.shape
    dt = texture.dtype
    w_full = jnp.concatenate(
        [jnp.ones((1, C), dt), weight.reshape(-1, C)], axis=0)
    b_full = jnp.concatenate(
        [jnp.zeros((1, C), dt), bias.reshape(-1, C)], axis=0)
    cam32 = cam.astype(jnp.int32)
    return pl.pallas_call(
        _cc_body,
        grid=(B // _IPB,),
        in_specs=[
            pl.BlockSpec(memory_space=pltpu.SMEM),
            pl.BlockSpec(memory_space=pltpu.SMEM),
            pl.BlockSpec(memory_space=pltpu.SMEM),
            pl.BlockSpec((_IPB, C, H, W), lambda i: (i, 0, 0, 0)),
        ],
        out_specs=pl.BlockSpec((_IPB, C, H, W), lambda i: (i, 0, 0, 0)),
        out_shape=jax.ShapeDtypeStruct(texture.shape, dt),
        compiler_params=pltpu.CompilerParams(
            dimension_semantics=("arbitrary",)),
    )(cam32, w_full, b_full, texture)
